# baseline (speedup 1.0000x reference)
import functools

import jax
import jax.numpy as jnp
from jax import lax
from jax.experimental import pallas as pl
from jax.experimental.pallas import tpu as pltpu
from jax.experimental.pallas import tpu_sc as plsc

N_NODES = 10000
N_EDGES = 320000
D = 128

NS = 16
EPW = N_EDGES // NS
B = 80
NCHUNK = EPW // B
NCT = N_EDGES // B
NB = 4
NE = 8
NGRP = (NCHUNK - 2) // NE
LAST = NCHUNK - 1
RPT = 624
TAIL = N_NODES - NS * RPT


def _spmm_sc(x, e_packed, w):
    mesh = plsc.VectorSubcoreMesh(core_axis_name="c", subcore_axis_name="s",
                                  num_cores=1)

    @functools.partial(
        pl.kernel,
        mesh=mesh,
        out_type=jax.ShapeDtypeStruct((N_NODES, D), jnp.float32),
        scratch_types=(
            [pltpu.VMEM((2, B), jnp.int32) for _ in range(NE)]
            + [pltpu.VMEM((B,), jnp.float32) for _ in range(NE)]
            + [pltpu.VMEM((B, D), jnp.float32) for _ in range(NB)]
            + [pltpu.VMEM_SHARED((N_NODES, D), jnp.float32)]
            + [pltpu.SemaphoreType.DMA for _ in range(NE)]
            + [pltpu.SemaphoreType.DMA for _ in range(NB)]
            + [pltpu.SemaphoreType.DMA for _ in range(NB)]
        ),
    )
    def spmm(x_hbm, e_hbm, w_hbm, out_hbm, *refs):
        ebuf = refs[0:NE]
        wbuf = refs[NE:2 * NE]
        rowsv = refs[2 * NE:2 * NE + NB]
        acc = refs[2 * NE + NB]
        sems = refs[2 * NE + NB + 1:]
        esem = sems[0:NE]
        gsem = sems[NE:NE + NB]
        ssem = sems[NE + NB:NE + 2 * NB]
        sid = lax.axis_index("s")

        zvec = jnp.zeros((16,), jnp.float32)

        def zero_body(r, carry):
            for j in range(D // 16):
                rowsv[0][r, pl.ds(j * 16, 16)] = zvec
            return carry

        lax.fori_loop(0, B, zero_body, 0)
        base = sid * RPT
        for k in range(RPT // B):
            pltpu.sync_copy(rowsv[0], acc.at[pl.ds(base + k * B, B)])
        rem = RPT - (RPT // B) * B
        pltpu.sync_copy(rowsv[0].at[pl.ds(0, rem)],
                        acc.at[pl.ds(base + RPT - rem, rem)])

        @pl.when(sid == NS - 1)
        def _zero_tail():
            pltpu.sync_copy(rowsv[0].at[pl.ds(0, TAIL)],
                            acc.at[pl.ds(NS * RPT, TAIL)])

        plsc.subcore_barrier()

        cbase = sid * NCHUNK

        def issue_e(ch, j):
            pltpu.async_copy(e_hbm.at[cbase + ch], ebuf[j], esem[j])
            off = (cbase + ch) * B
            pltpu.async_copy(w_hbm.at[pl.ds(off, B)], wbuf[j], esem[j])

        def wait_e(j):
            pltpu.make_async_copy(e_hbm.at[0], ebuf[j], esem[j]).wait()
            pltpu.make_async_copy(w_hbm.at[pl.ds(0, B)], wbuf[j],
                                  esem[j]).wait()

        H = B // 2

        def issue_gather(j, b):
            pltpu.async_copy(x_hbm.at[ebuf[j].at[0, pl.ds(0, H)]],
                             rowsv[b].at[pl.ds(0, H)], gsem[b])
            pltpu.async_copy(x_hbm.at[ebuf[j].at[0, pl.ds(H, H)]],
                             rowsv[b].at[pl.ds(H, H)], gsem[b])

        def wait_gather(j, b):
            pltpu.make_async_copy(x_hbm.at[ebuf[j].at[0, pl.ds(0, H)]],
                                  rowsv[b].at[pl.ds(0, H)], gsem[b]).wait()
            pltpu.make_async_copy(x_hbm.at[ebuf[j].at[0, pl.ds(H, H)]],
                                  rowsv[b].at[pl.ds(H, H)], gsem[b]).wait()

        def issue_scatter(j, b):
            pltpu.async_copy(rowsv[b], acc.at[ebuf[j].at[1]], ssem[b],
                             add=True)

        def wait_scatter(j, b):
            pltpu.make_async_copy(rowsv[b], acc.at[ebuf[j].at[1]],
                                  ssem[b]).wait()

        def scale(j, b):
            def scale_body(q, c2, _j=j, _b=b):
                wchunk = wbuf[_j][pl.ds(q * 16, 16)]
                for t in range(16):
                    r = q * 16 + t
                    wsv = jnp.take_along_axis(
                        wchunk, jnp.full((16,), t, jnp.int32), axis=0)
                    for f in range(D // 16):
                        sl = pl.ds(f * 16, 16)
                        rowsv[_b][r, sl] = rowsv[_b][r, sl] * wsv
                return c2

            lax.fori_loop(0, B // 16, scale_body, 0)

        def step(c, be, b, in_main):
            wait_gather(be, b)
            scale(be, b)
            issue_scatter(be, b)
            if in_main:
                @pl.when(c >= 2)
                def _drain():
                    wait_scatter((be + 6) % NE, (b + 2) % NB)

                @pl.when(c + 4 <= LAST)
                def _prefetch():
                    issue_e(c + 4, (be + 4) % NE)

                wait_e((be + 2) % NE)
                issue_gather((be + 2) % NE, (b + 2) % NB)
            return c

        for t in range(4):
            issue_e(t, t)
        for t in range(2):
            wait_e(t)
            issue_gather(t, t)

        def group_body(g, carry):
            for t in range(NE):
                step(NE * g + t, t, t % NB, True)
            return carry

        lax.fori_loop(0, NGRP, group_body, 0)

        for (c, be, b) in ((248, 0, 0), (249, 1, 1)):
            wait_gather(be, b)
            scale(be, b)
            issue_scatter(be, b)
            wait_scatter((be + 6) % NE, (b + 2) % NB)
        wait_scatter(0, 0)
        wait_scatter(1, 1)
        plsc.subcore_barrier()

        sl = pl.ds(sid * RPT, RPT)
        pltpu.sync_copy(acc.at[sl], out_hbm.at[sl])

        @pl.when(sid == NS - 1)
        def _write_tail():
            tl = pl.ds(NS * RPT, TAIL)
            pltpu.sync_copy(acc.at[tl], out_hbm.at[tl])

    return spmm(x, e_packed, w)


def _matmul_tc(h, W):
    BM = 2000
    dims = (((1,), (1,)), ((), ()))

    def body(h_ref, w_ref, o_ref):
        o_ref[...] = lax.dot_general(h_ref[...], w_ref[...], dims,
                                     preferred_element_type=jnp.float32)

    return pl.pallas_call(
        body,
        grid=(N_NODES // BM,),
        in_specs=[
            pl.BlockSpec((BM, D), lambda i: (i, 0)),
            pl.BlockSpec((D, D), lambda i: (0, 0)),
        ],
        out_specs=pl.BlockSpec((BM, D), lambda i: (i, 0)),
        out_shape=jax.ShapeDtypeStruct((N_NODES, D), jnp.float32),
    )(h, W)


def kernel(x, edge_index, edge_weight, W):
    row = edge_index[0].astype(jnp.int32)
    col = edge_index[1].astype(jnp.int32)
    e_packed = jnp.stack(
        [col.reshape(NCT, B), row.reshape(NCT, B)], axis=1)
    h = _spmm_sc(x, e_packed, edge_weight)
    return _matmul_tc(h, W)


# speedup vs baseline: 1.0003x; 1.0003x over previous
import functools

import jax
import jax.numpy as jnp
from jax import lax
from jax.experimental import pallas as pl
from jax.experimental.pallas import tpu as pltpu
from jax.experimental.pallas import tpu_sc as plsc

N_NODES = 10000
N_EDGES = 320000
D = 128

NS = 16
EPW = N_EDGES // NS
B = 80
NCHUNK = EPW // B
NCT = N_EDGES // B
NB = 4
NE = 8
NGRP = (NCHUNK - 2) // NE
LAST = NCHUNK - 1
RPT = 624
TAIL = N_NODES - NS * RPT


def _spmm_sc(x, e_packed, w):
    mesh = plsc.VectorSubcoreMesh(core_axis_name="c", subcore_axis_name="s",
                                  num_cores=1)

    @functools.partial(
        pl.kernel,
        mesh=mesh,
        out_type=jax.ShapeDtypeStruct((N_NODES, D), jnp.float32),
        scratch_types=(
            [pltpu.VMEM((2, B), jnp.int32) for _ in range(NE)]
            + [pltpu.VMEM((B,), jnp.float32) for _ in range(NE)]
            + [pltpu.VMEM((B, D), jnp.float32) for _ in range(NB)]
            + [pltpu.VMEM_SHARED((N_NODES, D), jnp.float32)]
            + [pltpu.SemaphoreType.DMA for _ in range(NE)]
            + [pltpu.SemaphoreType.DMA for _ in range(NB)]
            + [pltpu.SemaphoreType.DMA for _ in range(NB)]
        ),
    )
    def spmm(x_hbm, e_hbm, w_hbm, out_hbm, *refs):
        ebuf = refs[0:NE]
        wbuf = refs[NE:2 * NE]
        rowsv = refs[2 * NE:2 * NE + NB]
        acc = refs[2 * NE + NB]
        sems = refs[2 * NE + NB + 1:]
        esem = sems[0:NE]
        gsem = sems[NE:NE + NB]
        ssem = sems[NE + NB:NE + 2 * NB]
        sid = lax.axis_index("s")

        zvec = jnp.zeros((16,), jnp.float32)

        def zero_body(r, carry):
            for j in range(D // 16):
                rowsv[0][r, pl.ds(j * 16, 16)] = zvec
            return carry

        lax.fori_loop(0, B, zero_body, 0)
        base = sid * RPT
        for k in range(RPT // B):
            pltpu.sync_copy(rowsv[0], acc.at[pl.ds(base + k * B, B)])
        rem = RPT - (RPT // B) * B
        pltpu.sync_copy(rowsv[0].at[pl.ds(0, rem)],
                        acc.at[pl.ds(base + RPT - rem, rem)])

        @pl.when(sid == NS - 1)
        def _zero_tail():
            pltpu.sync_copy(rowsv[0].at[pl.ds(0, TAIL)],
                            acc.at[pl.ds(NS * RPT, TAIL)])

        plsc.subcore_barrier()

        cbase = sid * NCHUNK

        def issue_e(ch, j):
            pltpu.async_copy(e_hbm.at[cbase + ch], ebuf[j], esem[j])
            off = (cbase + ch) * B
            pltpu.async_copy(w_hbm.at[pl.ds(off, B)], wbuf[j], esem[j])

        def wait_e(j):
            pltpu.make_async_copy(e_hbm.at[0], ebuf[j], esem[j]).wait()
            pltpu.make_async_copy(w_hbm.at[pl.ds(0, B)], wbuf[j],
                                  esem[j]).wait()

        def issue_gather(j, b):
            pltpu.async_copy(x_hbm.at[ebuf[j].at[0]], rowsv[b], gsem[b])

        def wait_gather(j, b):
            pltpu.make_async_copy(x_hbm.at[ebuf[j].at[0]], rowsv[b],
                                  gsem[b]).wait()

        def issue_scatter(j, b):
            pltpu.async_copy(rowsv[b], acc.at[ebuf[j].at[1]], ssem[b],
                             add=True)

        def wait_scatter(j, b):
            pltpu.make_async_copy(rowsv[b], acc.at[ebuf[j].at[1]],
                                  ssem[b]).wait()

        def scale(j, b):
            def scale_body(q, c2, _j=j, _b=b):
                wchunk = wbuf[_j][pl.ds(q * 16, 16)]
                for t in range(16):
                    r = q * 16 + t
                    wsv = jnp.take_along_axis(
                        wchunk, jnp.full((16,), t, jnp.int32), axis=0)
                    for f in range(D // 16):
                        sl = pl.ds(f * 16, 16)
                        rowsv[_b][r, sl] = rowsv[_b][r, sl] * wsv
                return c2

            lax.fori_loop(0, B // 16, scale_body, 0)

        def step(c, be, b, in_main):
            wait_gather(be, b)
            scale(be, b)
            issue_scatter(be, b)
            if in_main:
                @pl.when(c >= 2)
                def _drain():
                    wait_scatter((be + 6) % NE, (b + 2) % NB)

                @pl.when(c + 4 <= LAST)
                def _prefetch():
                    issue_e(c + 4, (be + 4) % NE)

                wait_e((be + 2) % NE)
                issue_gather((be + 2) % NE, (b + 2) % NB)
            return c

        for t in range(4):
            issue_e(t, t)
        for t in range(2):
            wait_e(t)
            issue_gather(t, t)

        def group_body(g, carry):
            for t in range(NE):
                step(NE * g + t, t, t % NB, True)
            return carry

        lax.fori_loop(0, NGRP, group_body, 0)

        for (c, be, b) in ((248, 0, 0), (249, 1, 1)):
            wait_gather(be, b)
            scale(be, b)
            issue_scatter(be, b)
            wait_scatter((be + 6) % NE, (b + 2) % NB)
        wait_scatter(0, 0)
        wait_scatter(1, 1)
        plsc.subcore_barrier()

        sl = pl.ds(sid * RPT, RPT)
        pltpu.sync_copy(acc.at[sl], out_hbm.at[sl])

        @pl.when(sid == NS - 1)
        def _write_tail():
            tl = pl.ds(NS * RPT, TAIL)
            pltpu.sync_copy(acc.at[tl], out_hbm.at[tl])

    return spmm(x, e_packed, w)


def _matmul_tc(h, W):
    BM = 2000
    dims = (((1,), (1,)), ((), ()))

    def body(h_ref, w_ref, o_ref):
        o_ref[...] = lax.dot_general(h_ref[...], w_ref[...], dims,
                                     preferred_element_type=jnp.float32)

    return pl.pallas_call(
        body,
        grid=(N_NODES // BM,),
        in_specs=[
            pl.BlockSpec((BM, D), lambda i: (i, 0)),
            pl.BlockSpec((D, D), lambda i: (0, 0)),
        ],
        out_specs=pl.BlockSpec((BM, D), lambda i: (i, 0)),
        out_shape=jax.ShapeDtypeStruct((N_NODES, D), jnp.float32),
    )(h, W)


def kernel(x, edge_index, edge_weight, W):
    row = edge_index[0].astype(jnp.int32)
    col = edge_index[1].astype(jnp.int32)
    e_packed = jnp.stack(
        [col.reshape(NCT, B), row.reshape(NCT, B)], axis=1)
    h = _spmm_sc(x, e_packed, edge_weight)
    return _matmul_tc(h, W)


# speedup vs baseline: 1.0067x; 1.0064x over previous
import functools

import jax
import jax.numpy as jnp
from jax import lax
from jax.experimental import pallas as pl
from jax.experimental.pallas import tpu as pltpu
from jax.experimental.pallas import tpu_sc as plsc

N_NODES = 10000
N_EDGES = 320000
D = 128

NS = 16
EPW = N_EDGES // NS
B = 80
NCHUNK = EPW // B
NCT = N_EDGES // B
NB = 4
NE = 8
NGRP = (NCHUNK - 2) // NE
LAST = NCHUNK - 1
RPT = 624
TAIL = N_NODES - NS * RPT


def _spmm_sc(x, e_packed, w):
    mesh = plsc.VectorSubcoreMesh(core_axis_name="c", subcore_axis_name="s",
                                  num_cores=1)

    @functools.partial(
        pl.kernel,
        mesh=mesh,
        out_type=jax.ShapeDtypeStruct((N_NODES, D), jnp.float32),
        scratch_types=(
            [pltpu.VMEM((2, B), jnp.int32) for _ in range(NE)]
            + [pltpu.VMEM((B,), jnp.float32) for _ in range(NE)]
            + [pltpu.VMEM((B, D), jnp.float32) for _ in range(NB)]
            + [pltpu.VMEM_SHARED((N_NODES, D), jnp.float32)]
            + [pltpu.SemaphoreType.DMA for _ in range(NE)]
            + [pltpu.SemaphoreType.DMA for _ in range(NB)]
            + [pltpu.SemaphoreType.DMA for _ in range(NB)]
        ),
    )
    def spmm(x_hbm, e_hbm, w_hbm, out_hbm, *refs):
        ebuf = refs[0:NE]
        wbuf = refs[NE:2 * NE]
        rowsv = refs[2 * NE:2 * NE + NB]
        acc = refs[2 * NE + NB]
        sems = refs[2 * NE + NB + 1:]
        esem = sems[0:NE]
        gsem = sems[NE:NE + NB]
        ssem = sems[NE + NB:NE + 2 * NB]
        sid = lax.axis_index("s")

        zvec = jnp.zeros((16,), jnp.float32)

        def zero_body(r, carry):
            for j in range(D // 16):
                rowsv[0][r, pl.ds(j * 16, 16)] = zvec
            return carry

        lax.fori_loop(0, B, zero_body, 0)
        base = sid * RPT
        for k in range(RPT // B):
            pltpu.sync_copy(rowsv[0], acc.at[pl.ds(base + k * B, B)])
        rem = RPT - (RPT // B) * B
        pltpu.sync_copy(rowsv[0].at[pl.ds(0, rem)],
                        acc.at[pl.ds(base + RPT - rem, rem)])

        @pl.when(sid == NS - 1)
        def _zero_tail():
            pltpu.sync_copy(rowsv[0].at[pl.ds(0, TAIL)],
                            acc.at[pl.ds(NS * RPT, TAIL)])

        plsc.subcore_barrier()

        cbase = sid * NCHUNK

        def issue_e(ch, j):
            pltpu.async_copy(e_hbm.at[cbase + ch], ebuf[j], esem[j])
            off = (cbase + ch) * B
            pltpu.async_copy(w_hbm.at[pl.ds(off, B)], wbuf[j], esem[j])

        def wait_e(j):
            pltpu.make_async_copy(e_hbm.at[0], ebuf[j], esem[j]).wait()
            pltpu.make_async_copy(w_hbm.at[pl.ds(0, B)], wbuf[j],
                                  esem[j]).wait()

        def issue_gather(j, b):
            pltpu.async_copy(x_hbm.at[ebuf[j].at[0]], rowsv[b], gsem[b])

        def wait_gather(j, b):
            pltpu.make_async_copy(x_hbm.at[ebuf[j].at[0]], rowsv[b],
                                  gsem[b]).wait()

        def issue_scatter(j, b):
            pltpu.async_copy(rowsv[b], acc.at[ebuf[j].at[1]], ssem[b],
                             add=True)

        def wait_scatter(j, b):
            pltpu.make_async_copy(rowsv[b], acc.at[ebuf[j].at[1]],
                                  ssem[b]).wait()

        def scale(j, b):
            def scale_body(q, c2, _j=j, _b=b):
                wchunk = wbuf[_j][pl.ds(q * 16, 16)]
                for t in range(16):
                    r = q * 16 + t
                    wsv = jnp.take_along_axis(
                        wchunk, jnp.full((16,), t, jnp.int32), axis=0)
                    for f in range(D // 16):
                        sl = pl.ds(f * 16, 16)
                        rowsv[_b][r, sl] = rowsv[_b][r, sl] * wsv
                return c2

            lax.fori_loop(0, B // 16, scale_body, 0)

        def step(c, be, b, in_main):
            wait_gather(be, b)
            scale(be, b)
            issue_scatter(be, b)
            if in_main:
                @pl.when(c >= 2)
                def _drain():
                    wait_scatter((be + 6) % NE, (b + 2) % NB)

                @pl.when(c + 4 <= LAST)
                def _prefetch():
                    issue_e(c + 4, (be + 4) % NE)

                wait_e((be + 2) % NE)
                issue_gather((be + 2) % NE, (b + 2) % NB)
            return c

        for t in range(4):
            issue_e(t, t)
        for t in range(2):
            wait_e(t)
            issue_gather(t, t)

        def group_body(g, carry):
            for t in range(NE):
                step(NE * g + t, t, t % NB, True)
            return carry

        lax.fori_loop(0, NGRP, group_body, 0)

        for (c, be, b) in ((248, 0, 0), (249, 1, 1)):
            wait_gather(be, b)
            scale(be, b)
            issue_scatter(be, b)
            wait_scatter((be + 6) % NE, (b + 2) % NB)
        wait_scatter(0, 0)
        wait_scatter(1, 1)
        plsc.subcore_barrier()

        sl = pl.ds(sid * RPT, RPT)
        pltpu.sync_copy(acc.at[sl], out_hbm.at[sl])

        @pl.when(sid == NS - 1)
        def _write_tail():
            tl = pl.ds(NS * RPT, TAIL)
            pltpu.sync_copy(acc.at[tl], out_hbm.at[tl])

    return spmm(x, e_packed, w)


def _matmul_tc(h, W):
    BM = 10000
    dims = (((1,), (1,)), ((), ()))

    def body(h_ref, w_ref, o_ref):
        o_ref[...] = lax.dot_general(h_ref[...], w_ref[...], dims,
                                     preferred_element_type=jnp.float32)

    return pl.pallas_call(
        body,
        grid=(N_NODES // BM,),
        in_specs=[
            pl.BlockSpec((BM, D), lambda i: (i, 0)),
            pl.BlockSpec((D, D), lambda i: (0, 0)),
        ],
        out_specs=pl.BlockSpec((BM, D), lambda i: (i, 0)),
        out_shape=jax.ShapeDtypeStruct((N_NODES, D), jnp.float32),
    )(h, W)


def kernel(x, edge_index, edge_weight, W):
    row = edge_index[0].astype(jnp.int32)
    col = edge_index[1].astype(jnp.int32)
    e_packed = jnp.stack(
        [col.reshape(NCT, B), row.reshape(NCT, B)], axis=1)
    h = _spmm_sc(x, e_packed, edge_weight)
    return _matmul_tc(h, W)
